# Initial kernel scaffold; baseline (speedup 1.0000x reference)
import jax
import jax.numpy as jnp
from jax.experimental import pallas as pl


def kernel(x, edge_index, edge_attr, batch, W1, b1, g1, be1, W2, b2, g2, be2, W3, b3, g3, be3, Wm, bm, gm, bem):
    raise NotImplementedError("write your pallas kernel here")


# speedup vs baseline: 6.0620x; 6.0620x over previous
import functools

import jax
import jax.numpy as jnp
from jax import lax
from jax.experimental import pallas as pl
from jax.experimental.pallas import tpu as pltpu
from jax.experimental.pallas import tpu_sc as plsc

N = 10000
E = 320000
NG = 64
NP = 10240
NC, NS = 2, 16
NW = NC * NS
EB = 128
NBLK = E // EB
BN_EPS = 1e-5
_PREC = None

_mesh = plsc.VectorSubcoreMesh(core_axis_name="c", subcore_axis_name="s")


def _elu(t):
    return jnp.where(t > 0, t, jnp.exp(t) - 1.0)


def _bn(t, g, b):
    mu = jnp.mean(t, axis=0)
    var = jnp.mean((t - mu) ** 2, axis=0)
    return g * (t - mu) * lax.rsqrt(var + BN_EPS) + b


@functools.partial(
    pl.kernel,
    out_type=jax.ShapeDtypeStruct((NC * NP, 128), jnp.float32),
    mesh=_mesh,
    compiler_params=pltpu.CompilerParams(needs_layout_passes=False),
    scratch_types=[
        pltpu.VMEM((EB,), jnp.int32),
        pltpu.VMEM((EB,), jnp.float32),
        pltpu.VMEM((EB, 128), jnp.float32),
        pltpu.VMEM((64, 128), jnp.float32),
        pltpu.VMEM_SHARED((NP, 128), jnp.float32),
    ],
)
def _sc_deg(col_hbm, w_hbm, out_hbm, cidx_v, w_v, rows_v, z_v, sc_acc):
    c = lax.axis_index("c")
    s = lax.axis_index("s")
    wid = s * NC + c
    rows_per_tile = NP // NS
    zero16 = jnp.zeros((16,), jnp.float32)

    def _z(i, _):
        for r in range(8):
            z_v[i, pl.ds(16 * r, 16)] = zero16
        return 0
    lax.fori_loop(0, 64, _z, 0)

    def _zr(e, _):
        for r in range(8):
            rows_v[e, pl.ds(16 * r, 16)] = zero16
        return 0
    lax.fori_loop(0, EB, _zr, 0)

    def _z2(k, _):
        pltpu.sync_copy(z_v, sc_acc.at[pl.ds(rows_per_tile * s + 64 * k, 64)])
        return 0
    lax.fori_loop(0, rows_per_tile // 64, _z2, 0)
    plsc.subcore_barrier()

    lo = wid * NBLK // NW
    hi = (wid + 1) * NBLK // NW

    def _blk(b, _):
        base = b * EB
        pltpu.sync_copy(col_hbm.at[pl.ds(base, EB)], cidx_v)
        pltpu.sync_copy(w_hbm.at[pl.ds(base, EB)], w_v)

        def _bc(e, _):
            wb = plsc.load_gather(w_v, [jnp.zeros((16,), jnp.int32) + e])
            rows_v[e, pl.ds(0, 16)] = wb
            return 0
        lax.fori_loop(0, EB, _bc, 0)
        pltpu.sync_copy(rows_v, sc_acc.at[cidx_v], add=True)
        return 0
    lax.fori_loop(lo, hi, _blk, 0)
    plsc.subcore_barrier()

    def _wo(k, _):
        off = rows_per_tile * s + 64 * k
        pltpu.sync_copy(sc_acc.at[pl.ds(off, 64)],
                        out_hbm.at[pl.ds(c * NP + off, 64)])
        return 0
    lax.fori_loop(0, rows_per_tile // 64, _wo, 0)


def _make_sc_agg(feat_split):
    rows_per_tile = NP // NS
    DC = 128
    ysrc_rows = 2 * N if feat_split else N

    @functools.partial(
        pl.kernel,
        out_type=jax.ShapeDtypeStruct((NC * NP, DC), jnp.float32),
        mesh=_mesh,
        compiler_params=pltpu.CompilerParams(needs_layout_passes=False),
        scratch_types=[
            pltpu.VMEM((EB,), jnp.int32),
            pltpu.VMEM((EB,), jnp.int32),
            pltpu.VMEM((EB,), jnp.float32),
            pltpu.VMEM((EB, DC), jnp.float32),
            pltpu.VMEM((64, DC), jnp.float32),
            pltpu.VMEM_SHARED((NP, DC), jnp.float32),
            pltpu.SemaphoreType.DMA,
        ],
    )
    def _sc_agg(row_hbm, col_hbm, w_hbm, y_hbm, out_hbm,
                ridx_v, cidx_v, w_v, rows_v, z_v, acc_sh, sem):
        c = lax.axis_index("c")
        s = lax.axis_index("s")
        zero16 = jnp.zeros((16,), jnp.float32)

        def _z(i, _):
            for r in range(DC // 16):
                z_v[i, pl.ds(16 * r, 16)] = zero16
            return 0
        lax.fori_loop(0, 64, _z, 0)

        def _z2(k, _):
            pltpu.sync_copy(z_v, acc_sh.at[pl.ds(rows_per_tile * s + 64 * k, 64)])
            return 0
        lax.fori_loop(0, rows_per_tile // 64, _z2, 0)
        plsc.subcore_barrier()

        if feat_split:
            lo = s * NBLK // NS
            hi = (s + 1) * NBLK // NS
            coff = c * N
        else:
            wid = s * NC + c
            lo = wid * NBLK // NW
            hi = (wid + 1) * NBLK // NW
            coff = 0

        def _blk(b, _):
            base = b * EB
            pltpu.sync_copy(row_hbm.at[pl.ds(base, EB)], ridx_v)
            pltpu.sync_copy(col_hbm.at[pl.ds(base, EB)], cidx_v)
            pltpu.sync_copy(w_hbm.at[pl.ds(base, EB)], w_v)
            if feat_split:
                for i in range(EB // 16):
                    ridx_v[pl.ds(16 * i, 16)] = ridx_v[pl.ds(16 * i, 16)] + coff
            pltpu.async_copy(y_hbm.at[ridx_v], rows_v, sem).wait()

            def _scale(e, _):
                wb = plsc.load_gather(w_v, [jnp.zeros((16,), jnp.int32) + e])
                for r in range(DC // 16):
                    rows_v[e, pl.ds(16 * r, 16)] = (
                        rows_v[e, pl.ds(16 * r, 16)] * wb)
                return 0
            lax.fori_loop(0, EB, _scale, 0)

            pltpu.sync_copy(rows_v, acc_sh.at[cidx_v], add=True)
            return 0
        lax.fori_loop(lo, hi, _blk, 0)
        plsc.subcore_barrier()

        def _wo(k, _):
            off = rows_per_tile * s + 64 * k
            pltpu.sync_copy(acc_sh.at[pl.ds(off, 64)],
                            out_hbm.at[pl.ds(c * NP + off, 64)])
            return 0
        lax.fori_loop(0, rows_per_tile // 64, _wo, 0)

    return _sc_agg


_sc_agg_edge = _make_sc_agg(False)
_sc_agg_feat = _make_sc_agg(True)


def _cat(y):
    return y.reshape(N, 2, 128).transpose(1, 0, 2).reshape(2 * N, 128)


def _uncat(o):
    return (o.reshape(2, NP, 128)[:, :N, :]
            .transpose(1, 0, 2).reshape(N, 256))


NB = 400
NGRID = N // NB


def _blk(d2):
    return pl.BlockSpec((NB, d2), lambda i: (i, 0))


def _full(d1, d2):
    return pl.BlockSpec((d1, d2), lambda i: (0, 0))


def _tc_dinv_body(p0_ref, p1_ref, dinv_ref):
    deg = p0_ref[:, 0:1] + p1_ref[:, 0:1] + 1.0
    dinv_ref[...] = jnp.where(deg > 0, lax.rsqrt(deg), 0.0)


def _accum_stats(i, t, stats_ref):
    @pl.when(i == 0)
    def _init():
        stats_ref[...] = jnp.zeros_like(stats_ref)
    stats_ref[0:1, :] += jnp.sum(t, axis=0, keepdims=True)
    stats_ref[1:2, :] += jnp.sum(t * t, axis=0, keepdims=True)


def _bn_coeffs(stats, g, be):
    mu = stats[0:1, :] / N
    var = stats[1:2, :] / N - mu * mu
    a = g * lax.rsqrt(var + BN_EPS)
    return a, be - a * mu


def _tc_xw_body(h_ref, dinv_ref, W_ref, xw_ref, y_ref):
    xw = jnp.dot(h_ref[...], W_ref[...],
                 preferred_element_type=jnp.float32, precision=_PREC)
    xw_ref[...] = xw
    y_ref[...] = dinv_ref[...] * xw


def _xw_call(h, dinv, W, din, dout):
    return pl.pallas_call(
        _tc_xw_body,
        grid=(NGRID,),
        in_specs=[_blk(din), _blk(1), _full(din, dout)],
        out_specs=[_blk(dout), _blk(dout)],
        out_shape=[jax.ShapeDtypeStruct((N, dout), jnp.float32),
                   jax.ShapeDtypeStruct((N, dout), jnp.float32)],
    )(h, dinv, W)


def _tc_pre_body(agg_ref, xw_ref, dinv_ref, b_ref, t_ref, stats_ref):
    dinv = dinv_ref[...]
    t = dinv * agg_ref[...] + (dinv * dinv) * xw_ref[...] + b_ref[...]
    t_ref[...] = t
    _accum_stats(pl.program_id(0), t, stats_ref)


def _tc_pre_split_body(p0_ref, p1_ref, xw_ref, dinv_ref, b_ref,
                       t_ref, stats_ref):
    dinv = dinv_ref[...]
    t = (dinv * (p0_ref[...] + p1_ref[...])
         + (dinv * dinv) * xw_ref[...] + b_ref[...])
    t_ref[...] = t
    _accum_stats(pl.program_id(0), t, stats_ref)


def _pre_call(split, agg_args, xw, dinv, b, d):
    body = _tc_pre_split_body if split else _tc_pre_body
    in_specs = ([_blk(d)] * len(agg_args)
                + [_blk(d), _blk(1), _full(1, d)])
    return pl.pallas_call(
        body,
        grid=(NGRID,),
        in_specs=in_specs,
        out_specs=[_blk(d), _full(8, d)],
        out_shape=[jax.ShapeDtypeStruct((N, d), jnp.float32),
                   jax.ShapeDtypeStruct((8, d), jnp.float32)],
    )(*agg_args, xw, dinv, b)


def _tc_bnapply_mm_body(t_ref, stats_ref, dinv_ref, g_ref, be_ref, W_ref,
                        xw_ref, y_ref):
    a, bb = _bn_coeffs(stats_ref[...], g_ref[...], be_ref[...])
    h = _elu(a * t_ref[...] + bb)
    xw = jnp.dot(h, W_ref[...],
                 preferred_element_type=jnp.float32, precision=_PREC)
    xw_ref[...] = xw
    y_ref[...] = dinv_ref[...] * xw


def _bnapply_mm_call(t, stats, dinv, g, be, W, din, dout):
    return pl.pallas_call(
        _tc_bnapply_mm_body,
        grid=(NGRID,),
        in_specs=[_blk(din), _full(8, din), _blk(1),
                  _full(1, din), _full(1, din), _full(din, dout)],
        out_specs=[_blk(dout), _blk(dout)],
        out_shape=[jax.ShapeDtypeStruct((N, dout), jnp.float32),
                   jax.ShapeDtypeStruct((N, dout), jnp.float32)],
    )(t, stats, dinv, g, be, W)


def _tc_pool_body(t3_ref, stats_ref, g3_ref, be3_ref, bcol_ref,
                  sums_ref, counts_ref, maxs_ref):
    i = pl.program_id(0)
    a, bb = _bn_coeffs(stats_ref[...], g3_ref[...], be3_ref[...])
    h3 = _elu(a * t3_ref[...] + bb)

    bcol = bcol_ref[...]
    gids_row = lax.broadcasted_iota(jnp.int32, (1, NG), 1)
    onehotf = (bcol == gids_row).astype(jnp.float32)

    @pl.when(i == 0)
    def _init():
        sums_ref[...] = jnp.zeros_like(sums_ref)
        counts_ref[...] = jnp.zeros_like(counts_ref)
        maxs_ref[...] = jnp.full_like(maxs_ref, -2.0)

    dnums = (((0,), (0,)), ((), ()))
    sums_ref[...] += lax.dot_general(
        onehotf, h3, dnums, preferred_element_type=jnp.float32, precision=_PREC)
    counts_ref[...] += lax.dot_general(
        onehotf, jnp.ones((NB, 1), jnp.float32), dnums,
        preferred_element_type=jnp.float32, precision=_PREC)

    mx_rows = []
    for g in range(NG):
        filled = jnp.where(bcol == g, h3, -2.0)
        mx_rows.append(jnp.max(filled, axis=0)[None, :])
    maxs_ref[...] = jnp.maximum(maxs_ref[...], jnp.concatenate(mx_rows, axis=0))


def _tc_final_body(sums_ref, counts_ref, maxs_ref, Wm_ref, bm_ref,
                   gm_ref, bem_ref, emb_ref):
    counts = counts_ref[...]
    x_mean = sums_ref[...] / jnp.maximum(counts, 1.0)
    x_max = maxs_ref[...]
    x_max = jnp.where(x_max == -2.0, 0.0, x_max)
    out = jnp.concatenate([x_mean, x_max], axis=1)
    t = jnp.dot(out, Wm_ref[...], preferred_element_type=jnp.float32, precision=_PREC) + bm_ref[...]
    mu = jnp.mean(t, axis=0)
    var = jnp.mean((t - mu) ** 2, axis=0)
    bn = gm_ref[...] * (t - mu) * lax.rsqrt(var + BN_EPS) + bem_ref[...]
    emb_ref[...] = jnp.maximum(bn, 0.0)


def kernel(x, edge_index, edge_attr, batch,
           W1, b1, g1, be1, W2, b2, g2, be2, W3, b3, g3, be3,
           Wm, bm, gm, bem):
    f32 = jnp.float32
    row = edge_index[0].astype(jnp.int32)
    col = edge_index[1].astype(jnp.int32)
    w = edge_attr.astype(f32)

    degp = _sc_deg(col, w)
    dinvp = pl.pallas_call(
        _tc_dinv_body,
        out_shape=jax.ShapeDtypeStruct((NP, 1), f32))(degp[:NP], degp[NP:])
    dinv = dinvp[:N]

    xw1, y1 = _xw_call(x, dinv, W1, 128, 256)
    agg1 = _uncat(_sc_agg_feat(row, col, w, _cat(y1)))
    t1, stats1 = _pre_call(False, (agg1,), xw1, dinv, b1.reshape(1, 256), 256)

    xw2, y2 = _bnapply_mm_call(t1, stats1, dinv, g1.reshape(1, 256),
                               be1.reshape(1, 256), W2, 256, 256)
    agg2 = _uncat(_sc_agg_feat(row, col, w, _cat(y2)))
    t2, stats2 = _pre_call(False, (agg2,), xw2, dinv, b2.reshape(1, 256), 256)

    xw3, y3 = _bnapply_mm_call(t2, stats2, dinv, g2.reshape(1, 256),
                               be2.reshape(1, 256), W3, 256, 128)
    agg3p = _sc_agg_edge(row, col, w, y3)
    t3, stats3 = _pre_call(True, (agg3p[:N], agg3p[NP:NP + N]), xw3, dinv,
                           b3.reshape(1, 128), 128)

    bcol = batch.astype(jnp.int32).reshape(N, 1)
    sums, counts, maxs = pl.pallas_call(
        _tc_pool_body,
        grid=(NGRID,),
        in_specs=[_blk(128), _full(8, 128), _full(1, 128), _full(1, 128),
                  _blk(1)],
        out_specs=[_full(NG, 128), _full(NG, 1), _full(NG, 128)],
        out_shape=[jax.ShapeDtypeStruct((NG, 128), f32),
                   jax.ShapeDtypeStruct((NG, 1), f32),
                   jax.ShapeDtypeStruct((NG, 128), f32)],
    )(t3, stats3, g3.reshape(1, 128), be3.reshape(1, 128), bcol)

    emb = pl.pallas_call(
        _tc_final_body,
        out_shape=jax.ShapeDtypeStruct((NG, 128), f32),
    )(sums, counts, maxs, Wm, bm.reshape(1, 128), gm.reshape(1, 128),
      bem.reshape(1, 128))
    return emb
